# baseline (device time: 200236 ns/iter reference)
import jax
import jax.numpy as jnp
from jax import lax
from jax.experimental import pallas as pl
from jax.experimental.pallas import tpu as pltpu

N_DEV = 32
N_PL = 16
M = 2048
CH = M // N_PL
HCH = CH // 2
HN = 1024
P = 2
PCH = CH // P
XP = 4
XPCH = HCH // XP


def _coords_of_mesh(i):
    z = i // 8
    j = i % 8
    y = j // 2
    xb = j % 2
    x = jnp.where(y % 2 == 0, xb, 1 - xb)
    return x, y, z


def _mesh_of_coords(x, y, z):
    return z * 8 + y * 2 + jnp.where(y % 2 == 0, x, 1 - x)


def _q_of_yz(y, z):
    q_lo = 3 * z + jnp.where(z % 2 == 0, y - 1, 3 - y)
    return jnp.where(y == 0, 15 - z, q_lo)


def _yz_of_q(p):
    z_lo = p // 3
    r = p % 3
    y_lo = jnp.where(z_lo % 2 == 0, 1 + r, 3 - r)
    y = jnp.where(p < 12, y_lo, 0)
    z = jnp.where(p < 12, z_lo, 15 - p)
    return y, z


def kernel(x, w_mat):
    m, k_per = x.shape
    _, n = w_mat.shape

    def body(x_ref, w_ref, out_ref,
             initR, initL, rbufR, rbufL, xbufR, xbufL,
             ssR, rsR, ssL, rsL,
             xssR, xrsR, xssL, xrsL,
             gssR, grsR, gssL, grsL,
             agssR, agrsR, agssL, agrsL):
        my_mesh = lax.axis_index("i")
        xc, yc, zc = _coords_of_mesh(my_mesh)
        q = _q_of_yz(yc, zc)

        yR, zR = _yz_of_q(jnp.mod(q + 1, N_PL))
        yL, zL = _yz_of_q(jnp.mod(q + N_PL - 1, N_PL))
        right = _mesh_of_coords(xc, yR, zR)
        left = _mesh_of_coords(xc, yL, zL)
        partner = _mesh_of_coords(1 - xc, yc, zc)

        barrier_sem = pltpu.get_barrier_semaphore()
        for nbr in (left, right, partner):
            pl.semaphore_signal(
                barrier_sem, inc=1,
                device_id=(nbr,), device_id_type=pl.DeviceIdType.MESH,
            )
        pl.semaphore_wait(barrier_sem, 3)

        def partialA(c):
            xs = x_ref[pl.ds(c * CH, CH), :]
            return jnp.dot(xs, w_ref[:, :HN], preferred_element_type=jnp.float32)

        def partialB(c):
            xs = x_ref[pl.ds(c * CH, CH), :]
            return jnp.dot(xs, w_ref[:, HN:], preferred_element_type=jnp.float32)

        drain = []

        initR[:, :] = partialA(q)
        initL[:, :] = partialB(q)
        rsR_d = []
        rsL_d = []
        pA_prev = None
        pB_prev = None
        for s in range(N_PL - 1):
            curR = []
            curL = []
            for p in range(P):
                rows = pl.ds(p * PCH, PCH)
                if s > 0:
                    rsR_d[s - 1][p].wait_recv()
                    rbufR[s - 1, rows, :] = (
                        rbufR[s - 1, rows, :] + pA_prev[p * PCH:(p + 1) * PCH, :]
                    )
                srcR = initR.at[rows, :] if s == 0 else rbufR.at[s - 1, rows, :]
                dR = pltpu.make_async_remote_copy(
                    src_ref=srcR,
                    dst_ref=rbufR.at[s, rows, :],
                    send_sem=ssR.at[s * P + p], recv_sem=rsR.at[s * P + p],
                    device_id=(right,), device_id_type=pl.DeviceIdType.MESH,
                )
                dR.start()
                curR.append(dR)
                if s > 0:
                    rsL_d[s - 1][p].wait_recv()
                    rbufL[s - 1, rows, :] = (
                        rbufL[s - 1, rows, :] + pB_prev[p * PCH:(p + 1) * PCH, :]
                    )
                srcL = initL.at[rows, :] if s == 0 else rbufL.at[s - 1, rows, :]
                dL = pltpu.make_async_remote_copy(
                    src_ref=srcL,
                    dst_ref=rbufL.at[s, rows, :],
                    send_sem=ssL.at[s * P + p], recv_sem=rsL.at[s * P + p],
                    device_id=(left,), device_id_type=pl.DeviceIdType.MESH,
                )
                dL.start()
                curL.append(dL)
            rsR_d.append(curR)
            rsL_d.append(curL)
            drain.extend(curR)
            drain.extend(curL)
            pA_prev = partialA(jnp.mod(q - s - 1 + N_PL, N_PL))
            pB_prev = partialB(jnp.mod(q + s + 1, N_PL))

        last = N_PL - 2
        for p in range(P):
            rows = pl.ds(p * PCH, PCH)
            rsR_d[last][p].wait_recv()
            rbufR[last, rows, :] = (
                rbufR[last, rows, :] + pA_prev[p * PCH:(p + 1) * PCH, :]
            )
            rsL_d[last][p].wait_recv()
            rbufL[last, rows, :] = (
                rbufL[last, rows, :] + pB_prev[p * PCH:(p + 1) * PCH, :]
            )

        ownR = jnp.mod(q + 1, N_PL)
        ownL = jnp.mod(q + N_PL - 1, N_PL)
        keep = xc * HCH
        give = (1 - xc) * HCH
        xR_d = []
        xL_d = []
        for p in range(XP):
            dR = pltpu.make_async_remote_copy(
                src_ref=rbufR.at[last, pl.ds(give + p * XPCH, XPCH), :],
                dst_ref=xbufR.at[pl.ds(p * XPCH, XPCH), :],
                send_sem=xssR.at[p], recv_sem=xrsR.at[p],
                device_id=(partner,), device_id_type=pl.DeviceIdType.MESH,
            )
            dL = pltpu.make_async_remote_copy(
                src_ref=rbufL.at[last, pl.ds(give + p * XPCH, XPCH), :],
                dst_ref=xbufL.at[pl.ds(p * XPCH, XPCH), :],
                send_sem=xssL.at[p], recv_sem=xrsL.at[p],
                device_id=(partner,), device_id_type=pl.DeviceIdType.MESH,
            )
            dR.start()
            dL.start()
            xR_d.append(dR)
            xL_d.append(dL)
        drain.extend(xR_d)
        drain.extend(xL_d)

        rowsR = ownR * CH + keep
        rowsL = ownL * CH + keep
        gR_d = []
        gL_d = []
        for p in range(XP):
            prow = pl.ds(p * XPCH, XPCH)
            xR_d[p].wait_recv()
            redR = rbufR[last, pl.ds(keep + p * XPCH, XPCH), :] + xbufR[prow, :]
            out_ref[pl.ds(rowsR + p * XPCH, XPCH), :HN] = (
                redR * jax.nn.sigmoid(redR)
            )
            dgR = pltpu.make_async_remote_copy(
                src_ref=out_ref.at[pl.ds(rowsR + p * XPCH, XPCH), pl.ds(0, HN)],
                dst_ref=out_ref.at[pl.ds(rowsR + p * XPCH, XPCH), pl.ds(0, HN)],
                send_sem=gssR.at[p], recv_sem=grsR.at[p],
                device_id=(partner,), device_id_type=pl.DeviceIdType.MESH,
            )
            dgR.start()
            gR_d.append(dgR)

            xL_d[p].wait_recv()
            redL = rbufL[last, pl.ds(keep + p * XPCH, XPCH), :] + xbufL[prow, :]
            out_ref[pl.ds(rowsL + p * XPCH, XPCH), HN:] = (
                redL * jax.nn.sigmoid(redL)
            )
            dgL = pltpu.make_async_remote_copy(
                src_ref=out_ref.at[pl.ds(rowsL + p * XPCH, XPCH), pl.ds(HN, HN)],
                dst_ref=out_ref.at[pl.ds(rowsL + p * XPCH, XPCH), pl.ds(HN, HN)],
                send_sem=gssL.at[p], recv_sem=grsL.at[p],
                device_id=(partner,), device_id_type=pl.DeviceIdType.MESH,
            )
            dgL.start()
            gL_d.append(dgL)
        drain.extend(gR_d)
        drain.extend(gL_d)
        for p in range(XP):
            gR_d[p].wait_recv()
            gL_d[p].wait_recv()

        agR_d = []
        agL_d = []
        for s in range(N_PL - 1):
            cA = jnp.mod(q + 1 - s + N_PL, N_PL)
            cB = jnp.mod(q - 1 + s + N_PL, N_PL)
            curR = []
            curL = []
            for p in range(P):
                if s > 0:
                    agR_d[s - 1][p].wait_recv()
                    agL_d[s - 1][p].wait_recv()
                slA = (pl.ds(cA * CH + p * PCH, PCH), pl.ds(0, HN))
                slB = (pl.ds(cB * CH + p * PCH, PCH), pl.ds(HN, HN))
                aR = pltpu.make_async_remote_copy(
                    src_ref=out_ref.at[slA], dst_ref=out_ref.at[slA],
                    send_sem=agssR.at[s * P + p], recv_sem=agrsR.at[s * P + p],
                    device_id=(right,), device_id_type=pl.DeviceIdType.MESH,
                )
                aL = pltpu.make_async_remote_copy(
                    src_ref=out_ref.at[slB], dst_ref=out_ref.at[slB],
                    send_sem=agssL.at[s * P + p], recv_sem=agrsL.at[s * P + p],
                    device_id=(left,), device_id_type=pl.DeviceIdType.MESH,
                )
                aR.start()
                aL.start()
                curR.append(aR)
                curL.append(aL)
            agR_d.append(curR)
            agL_d.append(curL)
            drain.extend(curR)
            drain.extend(curL)
        for p in range(P):
            agR_d[N_PL - 2][p].wait_recv()
            agL_d[N_PL - 2][p].wait_recv()

        for r in drain:
            r.wait_send()

    nsem = (N_PL - 1) * P
    return pl.pallas_call(
        body,
        out_shape=jax.ShapeDtypeStruct((M, n), jnp.float32),
        in_specs=[
            pl.BlockSpec(memory_space=pltpu.VMEM),
            pl.BlockSpec(memory_space=pltpu.VMEM),
        ],
        out_specs=pl.BlockSpec(memory_space=pltpu.VMEM),
        scratch_shapes=[
            pltpu.VMEM((CH, HN), jnp.float32),
            pltpu.VMEM((CH, HN), jnp.float32),
            pltpu.VMEM((N_PL - 1, CH, HN), jnp.float32),
            pltpu.VMEM((N_PL - 1, CH, HN), jnp.float32),
            pltpu.VMEM((HCH, HN), jnp.float32),
            pltpu.VMEM((HCH, HN), jnp.float32),
            pltpu.SemaphoreType.DMA((nsem,)),
            pltpu.SemaphoreType.DMA((nsem,)),
            pltpu.SemaphoreType.DMA((nsem,)),
            pltpu.SemaphoreType.DMA((nsem,)),
            pltpu.SemaphoreType.DMA((XP,)),
            pltpu.SemaphoreType.DMA((XP,)),
            pltpu.SemaphoreType.DMA((XP,)),
            pltpu.SemaphoreType.DMA((XP,)),
            pltpu.SemaphoreType.DMA((XP,)),
            pltpu.SemaphoreType.DMA((XP,)),
            pltpu.SemaphoreType.DMA((XP,)),
            pltpu.SemaphoreType.DMA((XP,)),
            pltpu.SemaphoreType.DMA((nsem,)),
            pltpu.SemaphoreType.DMA((nsem,)),
            pltpu.SemaphoreType.DMA((nsem,)),
            pltpu.SemaphoreType.DMA((nsem,)),
        ],
        compiler_params=pltpu.CompilerParams(collective_id=0),
    )(x, w_mat)


# device time: 200041 ns/iter; 1.0010x vs baseline; 1.0010x over previous
import jax
import jax.numpy as jnp
from jax import lax
from jax.experimental import pallas as pl
from jax.experimental.pallas import tpu as pltpu

N_DEV = 32
N_PL = 16
M = 2048
CH = M // N_PL
HCH = CH // 2
HN = 1024
P = 2
PCH = CH // P
XP = 2
XPCH = HCH // XP


def _coords_of_mesh(i):
    z = i // 8
    j = i % 8
    y = j // 2
    xb = j % 2
    x = jnp.where(y % 2 == 0, xb, 1 - xb)
    return x, y, z


def _mesh_of_coords(x, y, z):
    return z * 8 + y * 2 + jnp.where(y % 2 == 0, x, 1 - x)


def _q_of_yz(y, z):
    q_lo = 3 * z + jnp.where(z % 2 == 0, y - 1, 3 - y)
    return jnp.where(y == 0, 15 - z, q_lo)


def _yz_of_q(p):
    z_lo = p // 3
    r = p % 3
    y_lo = jnp.where(z_lo % 2 == 0, 1 + r, 3 - r)
    y = jnp.where(p < 12, y_lo, 0)
    z = jnp.where(p < 12, z_lo, 15 - p)
    return y, z


def kernel(x, w_mat):
    m, k_per = x.shape
    _, n = w_mat.shape

    def body(x_ref, w_ref, out_ref,
             initR, initL, rbufR, rbufL, xbufR, xbufL,
             ssR, rsR, ssL, rsL,
             xssR, xrsR, xssL, xrsL,
             gssR, grsR, gssL, grsL,
             agssR, agrsR, agssL, agrsL):
        my_mesh = lax.axis_index("i")
        xc, yc, zc = _coords_of_mesh(my_mesh)
        q = _q_of_yz(yc, zc)

        yR, zR = _yz_of_q(jnp.mod(q + 1, N_PL))
        yL, zL = _yz_of_q(jnp.mod(q + N_PL - 1, N_PL))
        right = _mesh_of_coords(xc, yR, zR)
        left = _mesh_of_coords(xc, yL, zL)
        partner = _mesh_of_coords(1 - xc, yc, zc)

        barrier_sem = pltpu.get_barrier_semaphore()
        for nbr in (left, right, partner):
            pl.semaphore_signal(
                barrier_sem, inc=1,
                device_id=(nbr,), device_id_type=pl.DeviceIdType.MESH,
            )
        pl.semaphore_wait(barrier_sem, 3)

        def partialA(c):
            xs = x_ref[pl.ds(c * CH, CH), :]
            return jnp.dot(xs, w_ref[:, :HN], preferred_element_type=jnp.float32)

        def partialB(c):
            xs = x_ref[pl.ds(c * CH, CH), :]
            return jnp.dot(xs, w_ref[:, HN:], preferred_element_type=jnp.float32)

        drain = []

        initR[:, :] = partialA(q)
        initL[:, :] = partialB(q)
        rsR_d = []
        rsL_d = []
        pA_prev = None
        pB_prev = None
        for s in range(N_PL - 1):
            curR = []
            curL = []
            for p in range(P):
                rows = pl.ds(p * PCH, PCH)
                if s > 0:
                    rsR_d[s - 1][p].wait_recv()
                    rbufR[s - 1, rows, :] = (
                        rbufR[s - 1, rows, :] + pA_prev[p * PCH:(p + 1) * PCH, :]
                    )
                srcR = initR.at[rows, :] if s == 0 else rbufR.at[s - 1, rows, :]
                dR = pltpu.make_async_remote_copy(
                    src_ref=srcR,
                    dst_ref=rbufR.at[s, rows, :],
                    send_sem=ssR.at[s * P + p], recv_sem=rsR.at[s * P + p],
                    device_id=(right,), device_id_type=pl.DeviceIdType.MESH,
                )
                dR.start()
                curR.append(dR)
                if s > 0:
                    rsL_d[s - 1][p].wait_recv()
                    rbufL[s - 1, rows, :] = (
                        rbufL[s - 1, rows, :] + pB_prev[p * PCH:(p + 1) * PCH, :]
                    )
                srcL = initL.at[rows, :] if s == 0 else rbufL.at[s - 1, rows, :]
                dL = pltpu.make_async_remote_copy(
                    src_ref=srcL,
                    dst_ref=rbufL.at[s, rows, :],
                    send_sem=ssL.at[s * P + p], recv_sem=rsL.at[s * P + p],
                    device_id=(left,), device_id_type=pl.DeviceIdType.MESH,
                )
                dL.start()
                curL.append(dL)
            rsR_d.append(curR)
            rsL_d.append(curL)
            drain.extend(curR)
            drain.extend(curL)
            pA_prev = partialA(jnp.mod(q - s - 1 + N_PL, N_PL))
            pB_prev = partialB(jnp.mod(q + s + 1, N_PL))

        last = N_PL - 2
        for p in range(P):
            rows = pl.ds(p * PCH, PCH)
            rsR_d[last][p].wait_recv()
            rbufR[last, rows, :] = (
                rbufR[last, rows, :] + pA_prev[p * PCH:(p + 1) * PCH, :]
            )
            rsL_d[last][p].wait_recv()
            rbufL[last, rows, :] = (
                rbufL[last, rows, :] + pB_prev[p * PCH:(p + 1) * PCH, :]
            )

        ownR = jnp.mod(q + 1, N_PL)
        ownL = jnp.mod(q + N_PL - 1, N_PL)
        keep = xc * HCH
        give = (1 - xc) * HCH
        xR_d = []
        xL_d = []
        for p in range(XP):
            dR = pltpu.make_async_remote_copy(
                src_ref=rbufR.at[last, pl.ds(give + p * XPCH, XPCH), :],
                dst_ref=xbufR.at[pl.ds(p * XPCH, XPCH), :],
                send_sem=xssR.at[p], recv_sem=xrsR.at[p],
                device_id=(partner,), device_id_type=pl.DeviceIdType.MESH,
            )
            dL = pltpu.make_async_remote_copy(
                src_ref=rbufL.at[last, pl.ds(give + p * XPCH, XPCH), :],
                dst_ref=xbufL.at[pl.ds(p * XPCH, XPCH), :],
                send_sem=xssL.at[p], recv_sem=xrsL.at[p],
                device_id=(partner,), device_id_type=pl.DeviceIdType.MESH,
            )
            dR.start()
            dL.start()
            xR_d.append(dR)
            xL_d.append(dL)
        drain.extend(xR_d)
        drain.extend(xL_d)

        rowsR = ownR * CH + keep
        rowsL = ownL * CH + keep
        gR_d = []
        gL_d = []
        for p in range(XP):
            prow = pl.ds(p * XPCH, XPCH)
            xR_d[p].wait_recv()
            redR = rbufR[last, pl.ds(keep + p * XPCH, XPCH), :] + xbufR[prow, :]
            out_ref[pl.ds(rowsR + p * XPCH, XPCH), :HN] = (
                redR * jax.nn.sigmoid(redR)
            )
            dgR = pltpu.make_async_remote_copy(
                src_ref=out_ref.at[pl.ds(rowsR + p * XPCH, XPCH), pl.ds(0, HN)],
                dst_ref=out_ref.at[pl.ds(rowsR + p * XPCH, XPCH), pl.ds(0, HN)],
                send_sem=gssR.at[p], recv_sem=grsR.at[p],
                device_id=(partner,), device_id_type=pl.DeviceIdType.MESH,
            )
            dgR.start()
            gR_d.append(dgR)

            xL_d[p].wait_recv()
            redL = rbufL[last, pl.ds(keep + p * XPCH, XPCH), :] + xbufL[prow, :]
            out_ref[pl.ds(rowsL + p * XPCH, XPCH), HN:] = (
                redL * jax.nn.sigmoid(redL)
            )
            dgL = pltpu.make_async_remote_copy(
                src_ref=out_ref.at[pl.ds(rowsL + p * XPCH, XPCH), pl.ds(HN, HN)],
                dst_ref=out_ref.at[pl.ds(rowsL + p * XPCH, XPCH), pl.ds(HN, HN)],
                send_sem=gssL.at[p], recv_sem=grsL.at[p],
                device_id=(partner,), device_id_type=pl.DeviceIdType.MESH,
            )
            dgL.start()
            gL_d.append(dgL)
        drain.extend(gR_d)
        drain.extend(gL_d)
        for p in range(XP):
            gR_d[p].wait_recv()
            gL_d[p].wait_recv()

        agR_d = []
        agL_d = []
        for s in range(N_PL - 1):
            cA = jnp.mod(q + 1 - s + N_PL, N_PL)
            cB = jnp.mod(q - 1 + s + N_PL, N_PL)
            curR = []
            curL = []
            for p in range(P):
                if s > 0:
                    agR_d[s - 1][p].wait_recv()
                    agL_d[s - 1][p].wait_recv()
                slA = (pl.ds(cA * CH + p * PCH, PCH), pl.ds(0, HN))
                slB = (pl.ds(cB * CH + p * PCH, PCH), pl.ds(HN, HN))
                aR = pltpu.make_async_remote_copy(
                    src_ref=out_ref.at[slA], dst_ref=out_ref.at[slA],
                    send_sem=agssR.at[s * P + p], recv_sem=agrsR.at[s * P + p],
                    device_id=(right,), device_id_type=pl.DeviceIdType.MESH,
                )
                aL = pltpu.make_async_remote_copy(
                    src_ref=out_ref.at[slB], dst_ref=out_ref.at[slB],
                    send_sem=agssL.at[s * P + p], recv_sem=agrsL.at[s * P + p],
                    device_id=(left,), device_id_type=pl.DeviceIdType.MESH,
                )
                aR.start()
                aL.start()
                curR.append(aR)
                curL.append(aL)
            agR_d.append(curR)
            agL_d.append(curL)
            drain.extend(curR)
            drain.extend(curL)
        for p in range(P):
            agR_d[N_PL - 2][p].wait_recv()
            agL_d[N_PL - 2][p].wait_recv()

        for r in drain:
            r.wait_send()

    nsem = (N_PL - 1) * P
    return pl.pallas_call(
        body,
        out_shape=jax.ShapeDtypeStruct((M, n), jnp.float32),
        in_specs=[
            pl.BlockSpec(memory_space=pltpu.VMEM),
            pl.BlockSpec(memory_space=pltpu.VMEM),
        ],
        out_specs=pl.BlockSpec(memory_space=pltpu.VMEM),
        scratch_shapes=[
            pltpu.VMEM((CH, HN), jnp.float32),
            pltpu.VMEM((CH, HN), jnp.float32),
            pltpu.VMEM((N_PL - 1, CH, HN), jnp.float32),
            pltpu.VMEM((N_PL - 1, CH, HN), jnp.float32),
            pltpu.VMEM((HCH, HN), jnp.float32),
            pltpu.VMEM((HCH, HN), jnp.float32),
            pltpu.SemaphoreType.DMA((nsem,)),
            pltpu.SemaphoreType.DMA((nsem,)),
            pltpu.SemaphoreType.DMA((nsem,)),
            pltpu.SemaphoreType.DMA((nsem,)),
            pltpu.SemaphoreType.DMA((XP,)),
            pltpu.SemaphoreType.DMA((XP,)),
            pltpu.SemaphoreType.DMA((XP,)),
            pltpu.SemaphoreType.DMA((XP,)),
            pltpu.SemaphoreType.DMA((XP,)),
            pltpu.SemaphoreType.DMA((XP,)),
            pltpu.SemaphoreType.DMA((XP,)),
            pltpu.SemaphoreType.DMA((XP,)),
            pltpu.SemaphoreType.DMA((nsem,)),
            pltpu.SemaphoreType.DMA((nsem,)),
            pltpu.SemaphoreType.DMA((nsem,)),
            pltpu.SemaphoreType.DMA((nsem,)),
        ],
        compiler_params=pltpu.CompilerParams(collective_id=0),
    )(x, w_mat)
